# baseline (speedup 1.0000x reference)
import jax
import jax.numpy as jnp
from jax.experimental import pallas as pl
from jax.experimental.pallas import tpu as pltpu

_B = 128
_PREV_COLS = 256 * 28 * 28
_CURR_COLS = 512 * 14 * 14

_BBLK = 8
_JBLK = 4096


def _count_pos_kernel(c_ref, w_ref):
    mask = jnp.where(c_ref[...] > 0.0, 1.0, 0.0)
    w_ref[...] = jnp.sum(mask, axis=1, keepdims=True)


def _weighted_reduce_kernel(p_ref, w_ref, o_ref):
    sel = jnp.where(p_ref[...] > 0.0, w_ref[...], 0.0)
    o_ref[...] = jnp.sum(sel, axis=0, keepdims=True)


def kernel(prev_x, curr_x):
    pv = prev_x.reshape(_B, _PREV_COLS)
    cv = curr_x.reshape(_B, _CURR_COLS)

    w = pl.pallas_call(
        _count_pos_kernel,
        grid=(_B // _BBLK,),
        in_specs=[pl.BlockSpec((_BBLK, _CURR_COLS), lambda i: (i, 0))],
        out_specs=pl.BlockSpec((_BBLK, 1), lambda i: (i, 0)),
        out_shape=jax.ShapeDtypeStruct((_B, 1), jnp.float32),
        compiler_params=pltpu.CompilerParams(
            dimension_semantics=("parallel",),
        ),
    )(cv)

    out = pl.pallas_call(
        _weighted_reduce_kernel,
        grid=(_PREV_COLS // _JBLK,),
        in_specs=[
            pl.BlockSpec((_B, _JBLK), lambda j: (0, j)),
            pl.BlockSpec((_B, 1), lambda j: (0, 0)),
        ],
        out_specs=pl.BlockSpec((1, _JBLK), lambda j: (0, j)),
        out_shape=jax.ShapeDtypeStruct((1, _PREV_COLS), jnp.float32),
        compiler_params=pltpu.CompilerParams(
            dimension_semantics=("parallel",),
        ),
    )(pv, w)

    return out.reshape(256, 28, 28)


# speedup vs baseline: 8.1718x; 8.1718x over previous
import jax
import jax.numpy as jnp
from jax.experimental import pallas as pl
from jax.experimental.pallas import tpu as pltpu

_B = 128
_CP = 256
_CC = 512
_HP = 28
_HC = 14


def _count_kernel(c_ref, w_ref):
    @pl.when(pl.program_id(0) == 0)
    def _():
        w_ref[...] = jnp.zeros_like(w_ref)

    m = jnp.where(c_ref[...] > 0.0, 1.0, 0.0)
    part = jnp.sum(m, axis=(0, 1))
    tot = jnp.sum(part, axis=1, keepdims=True)
    w_ref[...] += jnp.broadcast_to(tot, w_ref.shape)


def _weighted_reduce_kernel(p_ref, w_ref, o_ref):
    x = p_ref[...]
    sel = jnp.where(x > 0.0, w_ref[...][None, None], 0.0)
    o_ref[...] = jnp.sum(sel, axis=2)


def kernel(prev_x, curr_x):
    pv = jnp.transpose(prev_x, (2, 3, 0, 1))
    cv = jnp.transpose(curr_x, (2, 3, 0, 1))

    w = pl.pallas_call(
        _count_kernel,
        grid=(_HC,),
        in_specs=[pl.BlockSpec((1, _HC, _B, _CC), lambda i: (i, 0, 0, 0))],
        out_specs=pl.BlockSpec((_B, _CP), lambda i: (0, 0)),
        out_shape=jax.ShapeDtypeStruct((_B, _CP), jnp.float32),
        compiler_params=pltpu.CompilerParams(
            dimension_semantics=("arbitrary",),
        ),
    )(cv)

    out = pl.pallas_call(
        _weighted_reduce_kernel,
        grid=(_HP,),
        in_specs=[
            pl.BlockSpec((1, _HP, _B, _CP), lambda i: (i, 0, 0, 0)),
            pl.BlockSpec((_B, _CP), lambda i: (0, 0)),
        ],
        out_specs=pl.BlockSpec((1, _HP, _CP), lambda i: (i, 0, 0)),
        out_shape=jax.ShapeDtypeStruct((_HP, _HP, _CP), jnp.float32),
        compiler_params=pltpu.CompilerParams(
            dimension_semantics=("parallel",),
        ),
    )(pv, w)

    return jnp.transpose(out, (2, 0, 1))


# speedup vs baseline: 8.2748x; 1.0126x over previous
import jax
import jax.numpy as jnp
from jax.experimental import pallas as pl
from jax.experimental.pallas import tpu as pltpu

_B = 128
_CP = 256
_CC = 512
_HP = 28
_HC = 14


def _fused_kernel(c_ref, p_ref, o_ref, acc_ref):
    i = pl.program_id(0)

    @pl.when(i == 0)
    def _():
        acc_ref[...] = jnp.zeros_like(acc_ref)

    @pl.when(i < _HC)
    def _():
        m = jnp.where(c_ref[...] > 0.0, 1.0, 0.0)
        part = jnp.sum(m, axis=(0, 1))
        tot = jnp.sum(part, axis=1, keepdims=True)
        acc_ref[...] += jnp.broadcast_to(tot, acc_ref.shape)

    @pl.when(i >= _HC)
    def _():
        x = p_ref[...]
        sel = jnp.where(x > 0.0, acc_ref[...][None, None], 0.0)
        o_ref[...] = jnp.sum(sel, axis=2)


def kernel(prev_x, curr_x):
    pv = jnp.transpose(prev_x, (2, 3, 0, 1))
    cv = jnp.transpose(curr_x, (2, 3, 0, 1))

    out = pl.pallas_call(
        _fused_kernel,
        grid=(_HC + _HP,),
        in_specs=[
            pl.BlockSpec(
                (1, _HC, _B, _CC),
                lambda i: (jnp.minimum(i, _HC - 1), 0, 0, 0),
            ),
            pl.BlockSpec(
                (1, _HP, _B, _CP),
                lambda i: (jnp.clip(i - _HC, 0, _HP - 1), 0, 0, 0),
            ),
        ],
        out_specs=pl.BlockSpec(
            (1, _HP, _CP),
            lambda i: (jnp.clip(i - _HC, 0, _HP - 1), 0, 0),
        ),
        out_shape=jax.ShapeDtypeStruct((_HP, _HP, _CP), jnp.float32),
        scratch_shapes=[pltpu.VMEM((_B, _CP), jnp.float32)],
        compiler_params=pltpu.CompilerParams(
            dimension_semantics=("arbitrary",),
        ),
    )(cv, pv)

    return jnp.transpose(out, (2, 0, 1))


# speedup vs baseline: 9.8399x; 1.1891x over previous
import jax
import jax.numpy as jnp
from jax.experimental import pallas as pl
from jax.experimental.pallas import tpu as pltpu

_B = 128
_CP = 256
_CC = 512
_HP = 28
_HC = 14


_NC = _HC // 2
_NP = _HP // 2


def _fused_kernel(c_ref, p_ref, o_ref, acc_ref):
    i = pl.program_id(0)

    @pl.when(i == 0)
    def _():
        acc_ref[...] = jnp.zeros_like(acc_ref)

    @pl.when(i < _NC)
    def _():
        m = jnp.where(c_ref[...] > 0.0, 1.0, 0.0)
        part = jnp.sum(m, axis=(0, 1))
        tot = jnp.sum(part, axis=1, keepdims=True)
        acc_ref[...] += jnp.broadcast_to(tot, acc_ref.shape)

    @pl.when(i >= _NC)
    def _():
        x = p_ref[...]
        sel = jnp.where(x > 0.0, acc_ref[...][None, None], 0.0)
        o_ref[...] = jnp.sum(sel, axis=2)


def kernel(prev_x, curr_x):
    pv = jnp.transpose(prev_x, (2, 3, 0, 1))
    cv = jnp.transpose(curr_x, (2, 3, 0, 1))

    out = pl.pallas_call(
        _fused_kernel,
        grid=(_NC + _NP,),
        in_specs=[
            pl.BlockSpec(
                (2, _HC, _B, _CC),
                lambda i: (jnp.minimum(i, _NC - 1), 0, 0, 0),
            ),
            pl.BlockSpec(
                (2, _HP, _B, _CP),
                lambda i: (jnp.clip(i - _NC, 0, _NP - 1), 0, 0, 0),
            ),
        ],
        out_specs=pl.BlockSpec(
            (2, _HP, _CP),
            lambda i: (jnp.clip(i - _NC, 0, _NP - 1), 0, 0),
        ),
        out_shape=jax.ShapeDtypeStruct((_HP, _HP, _CP), jnp.float32),
        scratch_shapes=[pltpu.VMEM((_B, _CP), jnp.float32)],
        compiler_params=pltpu.CompilerParams(
            dimension_semantics=("arbitrary",),
            vmem_limit_bytes=50 * 1024 * 1024,
        ),
    )(cv, pv)

    return jnp.transpose(out, (2, 0, 1))
